# baseline (speedup 1.0000x reference)
import functools

import jax
import jax.numpy as jnp
from jax import lax
from jax.experimental import pallas as pl
from jax.experimental.pallas import tpu as pltpu
from jax.experimental.pallas import tpu_sc as plsc

VOCAB = 100000
EMBED_DIM = 64
BATCH = 4096
SEQ = 200

FOLD = 16
WROWS = VOCAB // FOLD
WCOLS = FOLD * EMBED_DIM
WBLK = 1280
NWBLK = 5

VPAD = VOCAB

NC, NS, L = 2, 16, 16
NW = NC * NS
BPW = BATCH // NW
NFULL = SEQ // L
TAIL = SEQ - NFULL * L


def _tv_body(table_ref, k_ref, b_ref, out_ref):
    t = table_ref[...]
    k = k_ref[...]
    v = jnp.dot(t, k, preferred_element_type=jnp.float32)
    out_ref[...] = v + b_ref[0, 0]


def _compute_tv(table2, K, b2d):
    return pl.pallas_call(
        _tv_body,
        grid=(NWBLK,),
        in_specs=[
            pl.BlockSpec((WBLK, WCOLS), lambda i: (i, 0)),
            pl.BlockSpec((WCOLS, FOLD), lambda i: (0, 0)),
            pl.BlockSpec((1, 1), lambda i: (0, 0)),
        ],
        out_specs=pl.BlockSpec((WBLK, FOLD), lambda i: (i, 0)),
        out_shape=jax.ShapeDtypeStruct((WROWS, FOLD), jnp.float32),
    )(table2, K, b2d)


def _sc_body(tv_hbm, x_hbm, out_hbm, tv_v, idx_v, out_v):
    wid = lax.axis_index("s") * NC + lax.axis_index("c")
    base = wid * BPW
    pltpu.sync_copy(x_hbm.at[pl.ds(base * SEQ, BPW * SEQ)], idx_v)
    pltpu.sync_copy(tv_hbm, tv_v)

    lanes = lax.iota(jnp.int32, L)
    for g in range(BPW // L):
        pos0 = (jnp.full((L,), g * L, jnp.int32) + lanes) * SEQ

        def jstep(j, carry):
            acc, pos = carry
            ii = plsc.load_gather(idx_v, [pos])
            acc = acc + plsc.load_gather(tv_v, [ii])
            return acc, pos + 1

        acc, _ = lax.fori_loop(
            0, SEQ, jstep, (jnp.zeros((L,), jnp.float32), pos0)
        )
        out_v[pl.ds(g * L, L)] = acc * (1.0 / SEQ)

    pltpu.sync_copy(out_v, out_hbm.at[pl.ds(base, BPW)])


@functools.partial(jax.jit, static_argnames=())
def _pool(tv, x):
    mesh = plsc.VectorSubcoreMesh(core_axis_name="c", subcore_axis_name="s")
    f = pl.kernel(
        _sc_body,
        out_type=jax.ShapeDtypeStruct((BATCH,), jnp.float32),
        mesh=mesh,
        scratch_types=[
            pltpu.VMEM((VPAD,), jnp.float32),
            pltpu.VMEM((BPW * SEQ,), jnp.int32),
            pltpu.VMEM((BPW,), jnp.float32),
        ],
        compiler_params=pltpu.CompilerParams(needs_layout_passes=False),
    )
    return f(tv, x)


def kernel(x, table, W, b):
    table2 = table.reshape(WROWS, WCOLS)
    K = jnp.kron(jnp.eye(FOLD, dtype=jnp.float32), W)
    tv = _compute_tv(table2, K, b.reshape(1, 1).astype(jnp.float32))
    tv = tv.reshape(VOCAB)
    xf = x.reshape(BATCH * SEQ).astype(jnp.int32)
    return _pool(tv, xf)


# speedup vs baseline: 1.0712x; 1.0712x over previous
import functools

import jax
import jax.numpy as jnp
from jax import lax
from jax.experimental import pallas as pl
from jax.experimental.pallas import tpu as pltpu
from jax.experimental.pallas import tpu_sc as plsc

VOCAB = 100000
EMBED_DIM = 64
BATCH = 4096
SEQ = 200

FOLD = 16
WROWS = VOCAB // FOLD
WCOLS = FOLD * EMBED_DIM
WBLK = 1280
NWBLK = 5

VPAD = VOCAB

NC, NS, L = 2, 16, 16
NW = NC * NS
BPW = BATCH // NW
NFULL = SEQ // L
TAIL = SEQ - NFULL * L


def _tv_body(table_ref, k_ref, b_ref, out_ref):
    t = table_ref[...]
    k = k_ref[...]
    v = jnp.dot(t, k, preferred_element_type=jnp.float32)
    out_ref[...] = v + b_ref[0, 0]


def _compute_tv(table2, K, b2d):
    return pl.pallas_call(
        _tv_body,
        grid=(NWBLK,),
        in_specs=[
            pl.BlockSpec((WBLK, WCOLS), lambda i: (i, 0)),
            pl.BlockSpec((WCOLS, FOLD), lambda i: (0, 0)),
            pl.BlockSpec((1, 1), lambda i: (0, 0)),
        ],
        out_specs=pl.BlockSpec((WBLK, FOLD), lambda i: (i, 0)),
        out_shape=jax.ShapeDtypeStruct((WROWS, FOLD), jnp.float32),
    )(table2, K, b2d)


def _sc_body(tv_hbm, x_hbm, out_hbm, tv_v, idx_v, out_v):
    wid = lax.axis_index("s") * NC + lax.axis_index("c")
    base = wid * BPW
    pltpu.sync_copy(x_hbm.at[pl.ds(base * SEQ, BPW * SEQ)], idx_v)
    pltpu.sync_copy(tv_hbm, tv_v)

    lanes = lax.iota(jnp.int32, L)
    UNROLL = 4
    for g in range(BPW // L):
        pos0 = (jnp.full((L,), g * L, jnp.int32) + lanes) * SEQ

        def jstep(j, carry):
            accs, pos = carry
            new = []
            for u in range(UNROLL):
                ii = plsc.load_gather(idx_v, [pos + u])
                new.append(accs[u] + plsc.load_gather(tv_v, [ii]))
            return tuple(new), pos + UNROLL

        accs, _ = lax.fori_loop(
            0, SEQ // UNROLL, jstep,
            (tuple(jnp.zeros((L,), jnp.float32) for _ in range(UNROLL)), pos0),
        )
        acc = (accs[0] + accs[1]) + (accs[2] + accs[3])
        out_v[pl.ds(g * L, L)] = acc * (1.0 / SEQ)

    pltpu.sync_copy(out_v, out_hbm.at[pl.ds(base, BPW)])


@functools.partial(jax.jit, static_argnames=())
def _pool(tv, x):
    mesh = plsc.VectorSubcoreMesh(core_axis_name="c", subcore_axis_name="s")
    f = pl.kernel(
        _sc_body,
        out_type=jax.ShapeDtypeStruct((BATCH,), jnp.float32),
        mesh=mesh,
        scratch_types=[
            pltpu.VMEM((VPAD,), jnp.float32),
            pltpu.VMEM((BPW * SEQ,), jnp.int32),
            pltpu.VMEM((BPW,), jnp.float32),
        ],
        compiler_params=pltpu.CompilerParams(needs_layout_passes=False),
    )
    return f(tv, x)


def kernel(x, table, W, b):
    table2 = table.reshape(WROWS, WCOLS)
    K = jnp.kron(jnp.eye(FOLD, dtype=jnp.float32), W)
    tv = _compute_tv(table2, K, b.reshape(1, 1).astype(jnp.float32))
    tv = tv.reshape(VOCAB)
    xf = x.reshape(BATCH * SEQ).astype(jnp.int32)
    return _pool(tv, xf)
